# Initial kernel scaffold; baseline (speedup 1.0000x reference)
import jax
import jax.numpy as jnp
from jax.experimental import pallas as pl


def kernel(p, n, oph, pmt, opf, i, e_hit_sp, e_oph_pmt, e_pmt_opf, e_sp_evt, e_opf_evt, e_evt_sp, e_sp_hit, e_evt_opf, e_opf_pmt, e_pmt_oph, params):
    raise NotImplementedError("write your pallas kernel here")


# speedup vs baseline: 1.9099x; 1.9099x over previous
import functools

import jax
import jax.numpy as jnp
from jax import lax
from jax.experimental import pallas as pl
from jax.experimental.pallas import tpu as pltpu
from jax.experimental.pallas import tpu_sc as plsc

F32 = jnp.float32
I32 = jnp.int32
B = 128
F = 16
N_HIT, N_SP, N_OPH, N_PMT, N_OPF, N_EVT = 100000, 30000, 50000, 5000, 200, 16


def _ru(x, m):
    return (x + m - 1) // m * m


@functools.lru_cache(maxsize=None)
def _edge_kernel(E, N_src, N_dst, s, ds):
    nch = s // F
    TLE = 32 if ds == 1 else 16
    ep = _ru(E, TLE * 2 * B) // TLE
    NB = ep // B
    NDP = _ru(N_dst + 1, 128 * ds)
    H = NDP // ds
    TR = H + 128
    RTT = TR // 16
    npass = 1 if ds == 1 else ds // 2
    ZB = 64

    mesh = plsc.VectorSubcoreMesh(core_axis_name="c", subcore_axis_name="s")

    def body(xch, srcm, dstm, bvec, avec, gmat, outT,
             sidx, didx, wv, didxb, bsc, asc, rows, vals, gv, zbuf, table):
        core = lax.axis_index("c")
        sub = lax.axis_index("s")
        tile = core * 16 + sub if ds == 1 else sub
        pltpu.sync_copy(srcm.at[tile], sidx)
        pltpu.sync_copy(dstm.at[tile], didx)

        z16 = jnp.zeros((16,), F32)

        def zb(i, carry):
            zbuf[i, pl.ds(0, 16)] = z16
            zbuf[i, pl.ds(16, 16)] = z16
            return carry

        lax.fori_loop(0, ZB, zb, 0)

        def pha(bi, carry):
            pltpu.sync_copy(bvec.at[sidx.at[bi]], bsc)
            pltpu.sync_copy(avec.at[didx.at[bi]], asc)
            for j in range(B // 16):
                sl = pl.ds(j * 16, 16)
                z = asc[sl] + bsc[sl]
                wv[bi, sl] = 1.0 / (1.0 + jnp.exp(-z))
            return carry

        lax.fori_loop(0, NB, pha, 0)

        for p in range(npass):
            lo = (2 * p + core) * H if ds > 1 else 0

            def chunk(c, carry):
                base = sub * RTT
                for k0 in range(RTT // ZB):
                    pltpu.sync_copy(zbuf, table.at[pl.ds(base + k0 * ZB, ZB)])
                rem = RTT % ZB
                if rem:
                    pltpu.sync_copy(zbuf.at[pl.ds(0, rem)],
                                    table.at[pl.ds(base + (RTT // ZB) * ZB, rem)])
                pltpu.sync_copy(gmat.at[c], gv)
                plsc.subcore_barrier()
                g16 = gv[...]
                xc = xch.at[c]

                def batch(bi, bcarry):
                    pltpu.sync_copy(xc.at[sidx.at[bi]], rows)
                    for j in range(B // 16):
                        sl = pl.ds(j * 16, 16)
                        dl = didx[bi, sl] - lo
                        ok = (dl >= 0) & (dl < H)
                        didxb[sl] = jnp.where(ok, dl, H)

                    def grp(j, gcarry):
                        wvec = wv[bi, pl.ds(j * 16, 16)]
                        for l in range(16):
                            e = j * 16 + l
                            m = rows[e] * wvec[l]
                            ex = jnp.exp(m - g16)
                            vals[e, pl.ds(0, 16)] = ex * m
                            vals[e, pl.ds(16, 16)] = ex
                        return gcarry

                    lax.fori_loop(0, B // 16, grp, 0)
                    pltpu.sync_copy(vals, table.at[didxb], add=True)
                    return bcarry

                lax.fori_loop(0, NB, batch, 0)
                plsc.subcore_barrier()
                pltpu.sync_copy(table.at[pl.ds(base, RTT)],
                                outT.at[core, p, c, pl.ds(base, RTT)])
                return carry

            lax.fori_loop(0, nch, chunk, 0)

    kfn = pl.kernel(
        body,
        out_type=jax.ShapeDtypeStruct((2, npass, nch, TR, 2 * F), F32),
        mesh=mesh,
        compiler_params=pltpu.CompilerParams(use_tc_tiling_on_sc=False),
        scratch_types=[
            pltpu.VMEM((NB, B), I32),
            pltpu.VMEM((NB, B), I32),
            pltpu.VMEM((NB, B), F32),
            pltpu.VMEM((B,), I32),
            pltpu.VMEM((B,), F32),
            pltpu.VMEM((B,), F32),
            pltpu.VMEM((B, F), F32),
            pltpu.VMEM((B, 2 * F), F32),
            pltpu.VMEM((F,), F32),
            pltpu.VMEM((ZB, 2 * F), F32),
            pltpu.VMEM_SHARED((TR, 2 * F), F32),
        ],
    )
    return kfn, TLE, NB, H, TR, nch


def _vec_prep(x, wrow, bias):
    N, d = x.shape
    R = 512
    G = -(-N // R)

    def body(xr, wr, br, out):
        out[...] = jnp.sum(xr[...] * wr[...], axis=1, keepdims=True) + br[...]

    return pl.pallas_call(
        body,
        grid=(G,),
        in_specs=[pl.BlockSpec((R, d), lambda i: (i, 0)),
                  pl.BlockSpec((1, d), lambda i: (0, 0)),
                  pl.BlockSpec((1, 1), lambda i: (0, 0))],
        out_specs=pl.BlockSpec((R, 1), lambda i: (i, 0)),
        out_shape=jax.ShapeDtypeStruct((N, 1), F32),
    )(x, wrow, jnp.reshape(bias, (1, 1)).astype(F32))


def _g_prep(x):
    N, s = x.shape
    R = 512
    G = -(-N // R)

    def body(xr, out):
        i = pl.program_id(0)
        rows = lax.broadcasted_iota(I32, (R, s), 0) + i * R
        m = jnp.where(rows < N, xr[...], -jnp.inf)
        bm = jnp.maximum(jnp.max(m, axis=0, keepdims=True), 0.0)

        @pl.when(i == 0)
        def _():
            out[...] = bm

        @pl.when(i > 0)
        def _():
            out[...] = jnp.maximum(out[...], bm)

    return pl.pallas_call(
        body,
        grid=(G,),
        in_specs=[pl.BlockSpec((R, s), lambda i: (i, 0))],
        out_specs=pl.BlockSpec((1, s), lambda i: (0, 0)),
        out_shape=jax.ShapeDtypeStruct((1, s), F32),
    )(x)


def _mish(x):
    return x * jnp.tanh(jax.nn.softplus(x))


def _post(T, x_dst, W1, b1, W2, b2, N_dst, s, ds, H):
    nch = s // F
    t = x_dst.shape[1]
    o = W2.shape[0]
    R = 128
    G = -(-N_dst // R)
    hpr = H // R

    def body(Tr, xd, w1, bb1, w2, bb2, out):
        Tb = Tr[...]
        if ds == 1:
            num = Tb[0, 0, :, :, :F] + Tb[1, 0, :, :, :F]
            den = Tb[0, 0, :, :, F:] + Tb[1, 0, :, :, F:]
        else:
            num = Tb[0, 0, :, :, :F]
            den = Tb[0, 0, :, :, F:]
        aggr = jnp.where(den > 0, num / jnp.maximum(den, 1e-38), 0.0)
        parts = [aggr[c] for c in range(nch)] + [xd[...]]
        A = jnp.concatenate(parts, axis=1)
        h = _mish(A @ w1[...] + bb1[...])
        out[...] = _mish(h @ w2[...] + bb2[...])

    if ds == 1:
        tspec = pl.BlockSpec((2, 1, nch, R, 2 * F), lambda i: (0, 0, 0, i, 0))
    else:
        tspec = pl.BlockSpec((1, 1, nch, R, 2 * F),
                             lambda i: ((i // hpr) % 2, (i // hpr) // 2,
                                        0, i % hpr, 0))
    st = s + t
    return pl.pallas_call(
        body,
        grid=(G,),
        in_specs=[
            tspec,
            pl.BlockSpec((R, t), lambda i: (i, 0)),
            pl.BlockSpec((st, o), lambda i: (0, 0)),
            pl.BlockSpec((1, o), lambda i: (0, 0)),
            pl.BlockSpec((o, o), lambda i: (0, 0)),
            pl.BlockSpec((1, o), lambda i: (0, 0)),
        ],
        out_specs=pl.BlockSpec((R, o), lambda i: (i, 0)),
        out_shape=jax.ShapeDtypeStruct((N_dst, o), F32),
    )(T, x_dst, W1, b1, W2, b2)


def _block(prm, x_src, x_dst, eidx, n_dst):
    N_src, s = x_src.shape
    t = x_dst.shape[1]
    E = eidx.shape[1]
    ds = 1
    while (_ru(n_dst + 1, 128 * ds) // ds + 128) * 2 * F * 4 > 4 * 1024 * 1024:
        ds *= 2
    kfn, TLE, NB, H, TR, nch = _edge_kernel(E, N_src, n_dst, s, ds)

    We, be = prm["We"], prm["be"]
    a = _vec_prep(x_dst, We[:t].T, jnp.zeros(()))[:, 0]
    b = _vec_prep(x_src, We[t:].T, be[0])[:, 0]
    avec = jnp.pad(a, (0, _ru(n_dst + 1, 8) - n_dst))
    bvec = jnp.pad(b, (0, _ru(N_src, 8) - N_src))
    gmat = _g_prep(x_src).reshape(nch, F)
    xch = jnp.transpose(x_src.reshape(N_src, nch, F), (1, 0, 2))

    src, dst = eidx[0], eidx[1]
    epad = TLE * NB * B - E
    srcm = jnp.concatenate([src, jnp.zeros((epad,), I32)]).reshape(TLE, NB, B)
    dstm = jnp.concatenate([dst, jnp.full((epad,), n_dst, I32)]).reshape(TLE, NB, B)

    T = kfn(xch, srcm, dstm, bvec, avec, gmat)
    return _post(T, x_dst, prm["W1"], prm["b1"].reshape(1, -1),
                 prm["W2"], prm["b2"].reshape(1, -1), n_dst, s, ds, H)


def kernel(p, n, oph, pmt, opf, i, e_hit_sp, e_oph_pmt, e_pmt_opf, e_sp_evt,
           e_opf_evt, e_evt_sp, e_sp_hit, e_evt_opf, e_opf_pmt, e_pmt_oph,
           params):
    n1 = _block(params["plane_to_nexus"], p, n, e_hit_sp, N_SP)
    pmt1 = _block(params["hit_to_pmt"], oph, pmt, e_oph_pmt, N_PMT)
    opf1 = _block(params["pmt_to_flash"], pmt1, opf, e_pmt_opf, N_OPF)
    i1 = _block(params["sp_to_evt"], n1, i, e_sp_evt, N_EVT) \
        + _block(params["opf_to_evt"], opf1, i, e_opf_evt, N_EVT)
    n2 = _block(params["evt_to_sp"], i1, n1, e_evt_sp, N_SP)
    p1 = _block(params["sp_to_hit"], n2, p, e_sp_hit, N_HIT)
    opf2 = _block(params["evt_to_opf"], i1, opf1, e_evt_opf, N_OPF)
    pmt2 = _block(params["opf_to_pmt"], opf2, pmt1, e_opf_pmt, N_PMT)
    oph1 = _block(params["pmt_to_oph"], pmt2, oph, e_pmt_oph, N_OPH)
    return (p1, n2, oph1, pmt2, opf2, i1)


# speedup vs baseline: 2.0162x; 1.0556x over previous
import functools

import jax
import jax.numpy as jnp
from jax import lax
from jax.experimental import pallas as pl
from jax.experimental.pallas import tpu as pltpu
from jax.experimental.pallas import tpu_sc as plsc

F32 = jnp.float32
I32 = jnp.int32
B = 128
F = 16
N_HIT, N_SP, N_OPH, N_PMT, N_OPF, N_EVT = 100000, 30000, 50000, 5000, 200, 16


def _ru(x, m):
    return (x + m - 1) // m * m


@functools.lru_cache(maxsize=None)
def _edge_kernel(E, N_src, N_dst, s, ds):
    nch = s // F
    TLE = 32 if ds == 1 else 16
    NS = 8 if -(-E // TLE) >= 8 * B else 2
    NV = 4 if NS == 8 else 2
    ep = max(NS * B, _ru(-(-E // TLE), NS * B))
    NB = ep // B
    NDP = _ru(N_dst + 1, 128 * ds)
    H = NDP // ds
    TR = H + 128
    RTT = TR // 16
    npass = 1 if ds == 1 else ds // 2
    ZB = 64

    mesh = plsc.VectorSubcoreMesh(core_axis_name="c", subcore_axis_name="s")

    def body(xch, edg, bvec, avec, gmat, outT,
             wv, ering, rows, vals, didxL, bsc, asc, gv, zbuf, table,
             seme, semg, sems, semz):
        core = lax.axis_index("c")
        sub = lax.axis_index("s")
        tile = core * 16 + sub if ds == 1 else sub
        exm = edg.at[tile]

        def exb_issue(bi, slot):
            pltpu.async_copy(exm.at[bi], ering.at[slot], seme[slot])

        def exb_wait(bi, slot):
            pltpu.make_async_copy(exm.at[bi], ering.at[slot],
                                  seme[slot]).wait()

        z16 = jnp.zeros((16,), F32)

        def zb(i, carry):
            zbuf[i, pl.ds(0, 16)] = z16
            zbuf[i, pl.ds(16, 16)] = z16
            return carry

        lax.fori_loop(0, ZB, zb, 0)

        def ab_issue(slot):
            pltpu.async_copy(bvec.at[ering.at[slot, 0]], bsc.at[slot],
                             semg[slot])
            pltpu.async_copy(avec.at[ering.at[slot, 1]], asc.at[slot],
                             semg[slot])

        def ab_wait(slot):
            pltpu.make_async_copy(bvec.at[ering.at[slot, 0]], bsc.at[slot],
                                  semg[slot]).wait()
            pltpu.make_async_copy(avec.at[ering.at[slot, 1]], asc.at[slot],
                                  semg[slot]).wait()

        for q in range(NS):
            exb_issue(q, q)
        for q in range(NS - 1):
            exb_wait(q, q)
            ab_issue(q)

        def pha(g, carry):
            for sslot in range(NS):
                bi = g * NS + sslot
                nb1 = bi + NS - 1
                s1 = (sslot + NS - 1) % NS

                @pl.when(nb1 < NB)
                def _():
                    exb_wait(nb1, s1)
                    ab_issue(s1)

                ab_wait(sslot)
                for j in range(B // 16):
                    sl = pl.ds(j * 16, 16)
                    z = asc[sslot, sl] + bsc[sslot, sl]
                    wv[bi, sl] = 1.0 / (1.0 + jnp.exp(-z))

                @pl.when(bi + NS < NB)
                def _():
                    exb_issue(bi + NS, sslot)
            return carry

        lax.fori_loop(0, NB // NS, pha, 0)

        for p in range(npass):
            lo = (2 * p + core) * H if ds > 1 else 0

            def chunk(c, carry):
                base = sub * RTT
                nz = -(-RTT // ZB)
                for k0 in range(nz):
                    zr = min(ZB, RTT - k0 * ZB)
                    pltpu.async_copy(zbuf.at[pl.ds(0, zr)],
                                     table.at[pl.ds(base + k0 * ZB, zr)],
                                     semz)
                pltpu.sync_copy(gmat.at[c], gv)
                for k0 in range(nz):
                    zr = min(ZB, RTT - k0 * ZB)
                    pltpu.make_async_copy(
                        zbuf.at[pl.ds(0, zr)],
                        table.at[pl.ds(base + k0 * ZB, zr)], semz).wait()
                plsc.subcore_barrier()
                g16 = gv[...]
                xc = xch.at[c]

                def row_issue(bi, slot):
                    pltpu.async_copy(xc.at[ering.at[slot, 0]],
                                     rows.at[slot], semg[slot])

                def row_wait(slot):
                    pltpu.make_async_copy(xc.at[ering.at[slot, 0]],
                                          rows.at[slot], semg[slot]).wait()

                def sc_desc(v):
                    return pltpu.make_async_copy(
                        vals.at[v], table.at[didxL.at[v]], sems[v])

                for q in range(NS):
                    exb_issue(q, q)
                for q in range(NS - 1):
                    exb_wait(q, q)
                    row_issue(q, q)

                def batch(g, bcarry):
                    for sslot in range(NS):
                        bi = g * NS + sslot
                        v = sslot % NV
                        nb1 = bi + NS - 1
                        s1 = (sslot + NS - 1) % NS

                        @pl.when(nb1 < NB)
                        def _():
                            exb_wait(nb1, s1)
                            row_issue(nb1, s1)

                        row_wait(sslot)

                        @pl.when(bi >= NV)
                        def _():
                            sc_desc(v).wait()

                        for j in range(B // 16):
                            sl = pl.ds(j * 16, 16)
                            dl = ering[sslot, 1, sl] - lo
                            ok = (dl >= 0) & (dl < H)
                            didxL[v, sl] = jnp.where(ok, dl, H)

                        def grp(j, gcarry):
                            wvec = wv[bi, pl.ds(j * 16, 16)]
                            for l in range(16):
                                e = j * 16 + l
                                m = rows[sslot, e] * wvec[l]
                                ex = jnp.exp(m - g16)
                                vals[v, e, pl.ds(0, 16)] = ex * m
                                vals[v, e, pl.ds(16, 16)] = ex
                            return gcarry

                        lax.fori_loop(0, B // 16, grp, 0)
                        pltpu.async_copy(vals.at[v], table.at[didxL.at[v]],
                                         sems[v], add=True)

                        @pl.when(bi + NS < NB)
                        def _():
                            exb_issue(bi + NS, sslot)
                    return bcarry

                lax.fori_loop(0, NB // NS, batch, 0)
                for v in range(NV):
                    sc_desc(v).wait()
                plsc.subcore_barrier()
                pltpu.sync_copy(table.at[pl.ds(base, RTT)],
                                outT.at[core, p, c, pl.ds(base, RTT)])
                return carry

            lax.fori_loop(0, nch, chunk, 0)

    kfn = pl.kernel(
        body,
        out_type=jax.ShapeDtypeStruct((2, npass, nch, TR, 2 * F), F32),
        mesh=mesh,
        compiler_params=pltpu.CompilerParams(use_tc_tiling_on_sc=False),
        scratch_types=[
            pltpu.VMEM((NB, B), F32),
            pltpu.VMEM((NS, 2, B), I32),
            pltpu.VMEM((NS, B, F), F32),
            pltpu.VMEM((NV, B, 2 * F), F32),
            pltpu.VMEM((NV, B), I32),
            pltpu.VMEM((NS, B), F32),
            pltpu.VMEM((NS, B), F32),
            pltpu.VMEM((F,), F32),
            pltpu.VMEM((ZB, 2 * F), F32),
            pltpu.VMEM_SHARED((TR, 2 * F), F32),
            [pltpu.SemaphoreType.DMA] * NS,
            [pltpu.SemaphoreType.DMA] * NS,
            [pltpu.SemaphoreType.DMA] * NV,
            pltpu.SemaphoreType.DMA,
        ],
    )
    return kfn, TLE, NB, H, TR, nch


def _vec_prep(x, wrow, bias):
    N, d = x.shape
    R = 512
    G = -(-N // R)

    def body(xr, wr, br, out):
        out[...] = jnp.sum(xr[...] * wr[...], axis=1, keepdims=True) + br[...]

    return pl.pallas_call(
        body,
        grid=(G,),
        in_specs=[pl.BlockSpec((R, d), lambda i: (i, 0)),
                  pl.BlockSpec((1, d), lambda i: (0, 0)),
                  pl.BlockSpec((1, 1), lambda i: (0, 0))],
        out_specs=pl.BlockSpec((R, 1), lambda i: (i, 0)),
        out_shape=jax.ShapeDtypeStruct((N, 1), F32),
    )(x, wrow, jnp.reshape(bias, (1, 1)).astype(F32))


def _g_prep(x):
    N, s = x.shape
    R = 512
    G = -(-N // R)

    def body(xr, out):
        i = pl.program_id(0)
        rows = lax.broadcasted_iota(I32, (R, s), 0) + i * R
        m = jnp.where(rows < N, xr[...], -jnp.inf)
        bm = jnp.maximum(jnp.max(m, axis=0, keepdims=True), 0.0)

        @pl.when(i == 0)
        def _():
            out[...] = bm

        @pl.when(i > 0)
        def _():
            out[...] = jnp.maximum(out[...], bm)

    return pl.pallas_call(
        body,
        grid=(G,),
        in_specs=[pl.BlockSpec((R, s), lambda i: (i, 0))],
        out_specs=pl.BlockSpec((1, s), lambda i: (0, 0)),
        out_shape=jax.ShapeDtypeStruct((1, s), F32),
    )(x)


def _mish(x):
    return x * jnp.tanh(jax.nn.softplus(x))


def _post(T, x_dst, W1, b1, W2, b2, N_dst, s, ds, H):
    nch = s // F
    t = x_dst.shape[1]
    o = W2.shape[0]
    R = 128
    G = -(-N_dst // R)
    hpr = H // R

    def body(Tr, xd, w1, bb1, w2, bb2, out):
        Tb = Tr[...]
        if ds == 1:
            num = Tb[0, 0, :, :, :F] + Tb[1, 0, :, :, :F]
            den = Tb[0, 0, :, :, F:] + Tb[1, 0, :, :, F:]
        else:
            num = Tb[0, 0, :, :, :F]
            den = Tb[0, 0, :, :, F:]
        aggr = jnp.where(den > 0, num / jnp.maximum(den, 1e-38), 0.0)
        parts = [aggr[c] for c in range(nch)] + [xd[...]]
        A = jnp.concatenate(parts, axis=1)
        h = _mish(A @ w1[...] + bb1[...])
        out[...] = _mish(h @ w2[...] + bb2[...])

    if ds == 1:
        tspec = pl.BlockSpec((2, 1, nch, R, 2 * F), lambda i: (0, 0, 0, i, 0))
    else:
        tspec = pl.BlockSpec((1, 1, nch, R, 2 * F),
                             lambda i: ((i // hpr) % 2, (i // hpr) // 2,
                                        0, i % hpr, 0))
    st = s + t
    return pl.pallas_call(
        body,
        grid=(G,),
        in_specs=[
            tspec,
            pl.BlockSpec((R, t), lambda i: (i, 0)),
            pl.BlockSpec((st, o), lambda i: (0, 0)),
            pl.BlockSpec((1, o), lambda i: (0, 0)),
            pl.BlockSpec((o, o), lambda i: (0, 0)),
            pl.BlockSpec((1, o), lambda i: (0, 0)),
        ],
        out_specs=pl.BlockSpec((R, o), lambda i: (i, 0)),
        out_shape=jax.ShapeDtypeStruct((N_dst, o), F32),
    )(T, x_dst, W1, b1, W2, b2)


def _block(prm, x_src, x_dst, eidx, n_dst):
    N_src, s = x_src.shape
    t = x_dst.shape[1]
    E = eidx.shape[1]
    ds = 1
    while (_ru(n_dst + 1, 128 * ds) // ds + 128) * 2 * F * 4 > 4 * 1024 * 1024:
        ds *= 2
    kfn, TLE, NB, H, TR, nch = _edge_kernel(E, N_src, n_dst, s, ds)

    We, be = prm["We"], prm["be"]
    a = _vec_prep(x_dst, We[:t].T, jnp.zeros(()))[:, 0]
    b = _vec_prep(x_src, We[t:].T, be[0])[:, 0]
    avec = jnp.pad(a, (0, _ru(n_dst + 1, 8) - n_dst))
    bvec = jnp.pad(b, (0, _ru(N_src, 8) - N_src))
    gmat = _g_prep(x_src).reshape(nch, F)
    xch = jnp.transpose(x_src.reshape(N_src, nch, F), (1, 0, 2))

    src, dst = eidx[0], eidx[1]
    epad = TLE * NB * B - E
    srcm = jnp.concatenate([src, jnp.zeros((epad,), I32)]).reshape(TLE, NB, 1, B)
    dstm = jnp.concatenate([dst, jnp.full((epad,), n_dst, I32)]).reshape(TLE, NB, 1, B)
    edg = jnp.concatenate([srcm, dstm], axis=2)

    T = kfn(xch, edg, bvec, avec, gmat)
    return _post(T, x_dst, prm["W1"], prm["b1"].reshape(1, -1),
                 prm["W2"], prm["b2"].reshape(1, -1), n_dst, s, ds, H)


def kernel(p, n, oph, pmt, opf, i, e_hit_sp, e_oph_pmt, e_pmt_opf, e_sp_evt,
           e_opf_evt, e_evt_sp, e_sp_hit, e_evt_opf, e_opf_pmt, e_pmt_oph,
           params):
    n1 = _block(params["plane_to_nexus"], p, n, e_hit_sp, N_SP)
    pmt1 = _block(params["hit_to_pmt"], oph, pmt, e_oph_pmt, N_PMT)
    opf1 = _block(params["pmt_to_flash"], pmt1, opf, e_pmt_opf, N_OPF)
    i1 = _block(params["sp_to_evt"], n1, i, e_sp_evt, N_EVT) \
        + _block(params["opf_to_evt"], opf1, i, e_opf_evt, N_EVT)
    n2 = _block(params["evt_to_sp"], i1, n1, e_evt_sp, N_SP)
    p1 = _block(params["sp_to_hit"], n2, p, e_sp_hit, N_HIT)
    opf2 = _block(params["evt_to_opf"], i1, opf1, e_evt_opf, N_OPF)
    pmt2 = _block(params["opf_to_pmt"], opf2, pmt1, e_opf_pmt, N_PMT)
    oph1 = _block(params["pmt_to_oph"], pmt2, oph, e_pmt_oph, N_OPH)
    return (p1, n2, oph1, pmt2, opf2, i1)


# speedup vs baseline: 2.0162x; 1.0000x over previous
import functools

import jax
import jax.numpy as jnp
from jax import lax
from jax.experimental import pallas as pl
from jax.experimental.pallas import tpu as pltpu
from jax.experimental.pallas import tpu_sc as plsc

F32 = jnp.float32
I32 = jnp.int32
B = 128
F = 16
N_HIT, N_SP, N_OPH, N_PMT, N_OPF, N_EVT = 100000, 30000, 50000, 5000, 200, 16


def _ru(x, m):
    return (x + m - 1) // m * m


@functools.lru_cache(maxsize=None)
def _edge_kernel(E, N_src, N_dst, s, ds):
    nch = s // F
    TLE = 32 if ds == 1 else 16
    NS = 8 if -(-E // TLE) >= 8 * B else 2
    NV = 4 if NS == 8 else 2
    ep = max(NS * B, _ru(-(-E // TLE), NS * B))
    NB = ep // B
    NDP = _ru(N_dst + 1, 128 * ds)
    H = NDP // ds
    TR = H + 128
    RTT = TR // 16
    npass = 1 if ds == 1 else ds // 2
    ZB = 64

    mesh = plsc.VectorSubcoreMesh(core_axis_name="c", subcore_axis_name="s")

    NBP = _ru(NB, 16)

    def body(xch, edg, bvec, avec, gmat, bndA, bndB, outT,
             wv, ering, rows, vals, didxL, bsc, asc, gv, zbuf, bbv,
             table, seme, semg, sems, semz):
        core = lax.axis_index("c")
        sub = lax.axis_index("s")
        tile = core * 16 + sub if ds == 1 else sub
        exm = edg.at[tile]

        def exb_issue(bi, slot):
            pltpu.async_copy(exm.at[bi], ering.at[slot], seme[slot])

        def exb_wait(bi, slot):
            pltpu.make_async_copy(exm.at[bi], ering.at[slot],
                                  seme[slot]).wait()

        z16 = jnp.zeros((16,), F32)

        def zb(i, carry):
            zbuf[i, pl.ds(0, 16)] = z16
            zbuf[i, pl.ds(16, 16)] = z16
            return carry

        lax.fori_loop(0, ZB, zb, 0)

        def ab_issue(slot):
            pltpu.async_copy(bvec.at[ering.at[slot, 0]], bsc.at[slot],
                             semg[slot])
            pltpu.async_copy(avec.at[ering.at[slot, 1]], asc.at[slot],
                             semg[slot])

        def ab_wait(slot):
            pltpu.make_async_copy(bvec.at[ering.at[slot, 0]], bsc.at[slot],
                                  semg[slot]).wait()
            pltpu.make_async_copy(avec.at[ering.at[slot, 1]], asc.at[slot],
                                  semg[slot]).wait()

        for q in range(NS):
            exb_issue(q, q)
        for q in range(NS - 1):
            exb_wait(q, q)
            ab_issue(q)

        def pha(g, carry):
            for sslot in range(NS):
                bi = g * NS + sslot
                nb1 = bi + NS - 1
                s1 = (sslot + NS - 1) % NS

                @pl.when(nb1 < NB)
                def _():
                    exb_wait(nb1, s1)
                    ab_issue(s1)

                ab_wait(sslot)
                for j in range(B // 16):
                    sl = pl.ds(j * 16, 16)
                    z = asc[sslot, sl] + bsc[sslot, sl]
                    wv[bi, sl] = 1.0 / (1.0 + jnp.exp(-z))

                @pl.when(bi + NS < NB)
                def _():
                    exb_issue(bi + NS, sslot)
            return carry

        lax.fori_loop(0, NB // NS, pha, 0)

        for p in range(npass):
            lo = (2 * p + core) * H if ds > 1 else 0
            anyb = None
            b0 = 0
            bend = NB

            def chunk(c, carry):
                base = sub * RTT
                nz = -(-RTT // ZB)
                for k0 in range(nz):
                    zr = min(ZB, RTT - k0 * ZB)
                    pltpu.async_copy(zbuf.at[pl.ds(0, zr)],
                                     table.at[pl.ds(base + k0 * ZB, zr)],
                                     semz)
                pltpu.sync_copy(gmat.at[c], gv)
                for k0 in range(nz):
                    zr = min(ZB, RTT - k0 * ZB)
                    pltpu.make_async_copy(
                        zbuf.at[pl.ds(0, zr)],
                        table.at[pl.ds(base + k0 * ZB, zr)], semz).wait()
                plsc.subcore_barrier()
                g16 = gv[...]
                xc = xch.at[c]

                def row_issue(bi, slot):
                    pltpu.async_copy(xc.at[ering.at[slot, 0]],
                                     rows.at[slot], semg[slot])

                def row_wait(slot):
                    pltpu.make_async_copy(xc.at[ering.at[slot, 0]],
                                          rows.at[slot], semg[slot]).wait()

                def sc_desc(v):
                    return pltpu.make_async_copy(
                        vals.at[v], table.at[didxL.at[v]], sems[v])

                def pipe():
                    for q in range(NS):
                        exb_issue(b0 + q, q)
                    for q in range(NS - 1):
                        exb_wait(b0 + q, q)
                        row_issue(b0 + q, q)

                    def batch(g, bcarry):
                        for sslot in range(NS):
                            bi = g * NS + sslot
                            v = sslot % NV
                            nb1 = bi + NS - 1
                            s1 = (sslot + NS - 1) % NS

                            @pl.when(nb1 < bend)
                            def _():
                                exb_wait(nb1, s1)
                                row_issue(nb1, s1)

                            row_wait(sslot)

                            @pl.when(bi >= b0 + NV)
                            def _():
                                sc_desc(v).wait()

                            for j in range(B // 16):
                                sl = pl.ds(j * 16, 16)
                                dl = ering[sslot, 1, sl] - lo
                                ok = (dl >= 0) & (dl < H)
                                didxL[v, sl] = jnp.where(ok, dl, H)

                            def grp(j, gcarry):
                                wvec = wv[bi, pl.ds(j * 16, 16)]
                                for l in range(16):
                                    e = j * 16 + l
                                    m = rows[sslot, e] * wvec[l]
                                    ex = jnp.exp(m - g16)
                                    vals[v, e, pl.ds(0, 16)] = ex * m
                                    vals[v, e, pl.ds(16, 16)] = ex
                                return gcarry

                            lax.fori_loop(0, B // 16, grp, 0)
                            pltpu.async_copy(vals.at[v],
                                             table.at[didxL.at[v]],
                                             sems[v], add=True)

                            @pl.when(bi + NS < bend)
                            def _():
                                exb_issue(bi + NS, sslot)
                        return bcarry

                    lax.fori_loop(b0 // NS, bend // NS, batch, 0)
                    for v in range(NV):
                        sc_desc(v).wait()

                pipe()
                plsc.subcore_barrier()
                pltpu.sync_copy(table.at[pl.ds(base, RTT)],
                                outT.at[core, p, c, pl.ds(base, RTT)])
                return carry

            lax.fori_loop(0, nch, chunk, 0)

    kfn = pl.kernel(
        body,
        out_type=jax.ShapeDtypeStruct((2, npass, nch, TR, 2 * F), F32),
        mesh=mesh,
        compiler_params=pltpu.CompilerParams(use_tc_tiling_on_sc=False),
        scratch_types=[
            pltpu.VMEM((NB, B), F32),
            pltpu.VMEM((NS, 2, B), I32),
            pltpu.VMEM((NS, B, F), F32),
            pltpu.VMEM((NV, B, 2 * F), F32),
            pltpu.VMEM((NV, B), I32),
            pltpu.VMEM((NS, B), F32),
            pltpu.VMEM((NS, B), F32),
            pltpu.VMEM((F,), F32),
            pltpu.VMEM((ZB, 2 * F), F32),
            pltpu.VMEM((2, NB + 16), I32),
            pltpu.VMEM_SHARED((TR, 2 * F), F32),
            [pltpu.SemaphoreType.DMA] * NS,
            [pltpu.SemaphoreType.DMA] * NS,
            [pltpu.SemaphoreType.DMA] * NV,
            pltpu.SemaphoreType.DMA,
        ],
    )
    return kfn, TLE, NB, H, TR, nch


def _vec_prep(x, wrow, bias):
    N, d = x.shape
    R = 512
    G = -(-N // R)

    def body(xr, wr, br, out):
        out[...] = jnp.sum(xr[...] * wr[...], axis=1, keepdims=True) + br[...]

    return pl.pallas_call(
        body,
        grid=(G,),
        in_specs=[pl.BlockSpec((R, d), lambda i: (i, 0)),
                  pl.BlockSpec((1, d), lambda i: (0, 0)),
                  pl.BlockSpec((1, 1), lambda i: (0, 0))],
        out_specs=pl.BlockSpec((R, 1), lambda i: (i, 0)),
        out_shape=jax.ShapeDtypeStruct((N, 1), F32),
    )(x, wrow, jnp.reshape(bias, (1, 1)).astype(F32))


def _g_prep(x):
    N, s = x.shape
    R = 512
    G = -(-N // R)

    def body(xr, out):
        i = pl.program_id(0)
        rows = lax.broadcasted_iota(I32, (R, s), 0) + i * R
        m = jnp.where(rows < N, xr[...], -jnp.inf)
        bm = jnp.maximum(jnp.max(m, axis=0, keepdims=True), 0.0)

        @pl.when(i == 0)
        def _():
            out[...] = bm

        @pl.when(i > 0)
        def _():
            out[...] = jnp.maximum(out[...], bm)

    return pl.pallas_call(
        body,
        grid=(G,),
        in_specs=[pl.BlockSpec((R, s), lambda i: (i, 0))],
        out_specs=pl.BlockSpec((1, s), lambda i: (0, 0)),
        out_shape=jax.ShapeDtypeStruct((1, s), F32),
    )(x)


def _mish(x):
    return x * jnp.tanh(jax.nn.softplus(x))


def _post(T, x_dst, W1, b1, W2, b2, N_dst, s, ds, H):
    nch = s // F
    t = x_dst.shape[1]
    o = W2.shape[0]
    R = 128
    G = -(-N_dst // R)
    hpr = H // R

    def body(Tr, xd, w1, bb1, w2, bb2, out):
        Tb = Tr[...]
        if ds == 1:
            num = Tb[0, 0, :, :, :F] + Tb[1, 0, :, :, :F]
            den = Tb[0, 0, :, :, F:] + Tb[1, 0, :, :, F:]
        else:
            num = Tb[0, 0, :, :, :F]
            den = Tb[0, 0, :, :, F:]
        aggr = jnp.where(den > 0, num / jnp.maximum(den, 1e-38), 0.0)
        parts = [aggr[c] for c in range(nch)] + [xd[...]]
        A = jnp.concatenate(parts, axis=1)
        h = _mish(A @ w1[...] + bb1[...])
        out[...] = _mish(h @ w2[...] + bb2[...])

    if ds == 1:
        tspec = pl.BlockSpec((2, 1, nch, R, 2 * F), lambda i: (0, 0, 0, i, 0))
    else:
        tspec = pl.BlockSpec((1, 1, nch, R, 2 * F),
                             lambda i: ((i // hpr) % 2, (i // hpr) // 2,
                                        0, i % hpr, 0))
    st = s + t
    return pl.pallas_call(
        body,
        grid=(G,),
        in_specs=[
            tspec,
            pl.BlockSpec((R, t), lambda i: (i, 0)),
            pl.BlockSpec((st, o), lambda i: (0, 0)),
            pl.BlockSpec((1, o), lambda i: (0, 0)),
            pl.BlockSpec((o, o), lambda i: (0, 0)),
            pl.BlockSpec((1, o), lambda i: (0, 0)),
        ],
        out_specs=pl.BlockSpec((R, o), lambda i: (i, 0)),
        out_shape=jax.ShapeDtypeStruct((N_dst, o), F32),
    )(T, x_dst, W1, b1, W2, b2)


def _block(prm, x_src, x_dst, eidx, n_dst):
    N_src, s = x_src.shape
    t = x_dst.shape[1]
    E = eidx.shape[1]
    ds = 1
    while (_ru(n_dst + 1, 128 * ds) // ds + 128) * 2 * F * 4 > 4 * 1024 * 1024:
        ds *= 2
    kfn, TLE, NB, H, TR, nch = _edge_kernel(E, N_src, n_dst, s, ds)

    We, be = prm["We"], prm["be"]
    a = _vec_prep(x_dst, We[:t].T, jnp.zeros(()))[:, 0]
    b = _vec_prep(x_src, We[t:].T, be[0])[:, 0]
    avec = jnp.pad(a, (0, _ru(n_dst + 1, 8) - n_dst))
    bvec = jnp.pad(b, (0, _ru(N_src, 8) - N_src))
    gmat = _g_prep(x_src).reshape(nch, F)
    xch = jnp.transpose(x_src.reshape(N_src, nch, F), (1, 0, 2))

    src, dst = eidx[0], eidx[1]
    epad = TLE * NB * B - E
    srcp = jnp.concatenate([src, jnp.zeros((epad,), I32)])
    dstp = jnp.concatenate([dst, jnp.full((epad,), n_dst, I32)])
    edg = jnp.concatenate([srcp.reshape(TLE, NB, 1, B),
                           dstp.reshape(TLE, NB, 1, B)], axis=2)
    if ds > 1:
        bnd = dstp[::B]
        bndA = bnd.reshape(TLE, NB)
        bndB = jnp.concatenate([bnd[1:], jnp.full((1,), n_dst, I32)]
                               ).reshape(TLE, NB)
    else:
        bndA = bndB = jnp.zeros((TLE, NB), I32)

    T = kfn(xch, edg, bvec, avec, gmat, bndA, bndB)
    return _post(T, x_dst, prm["W1"], prm["b1"].reshape(1, -1),
                 prm["W2"], prm["b2"].reshape(1, -1), n_dst, s, ds, H)


def kernel(p, n, oph, pmt, opf, i, e_hit_sp, e_oph_pmt, e_pmt_opf, e_sp_evt,
           e_opf_evt, e_evt_sp, e_sp_hit, e_evt_opf, e_opf_pmt, e_pmt_oph,
           params):
    n1 = _block(params["plane_to_nexus"], p, n, e_hit_sp, N_SP)
    pmt1 = _block(params["hit_to_pmt"], oph, pmt, e_oph_pmt, N_PMT)
    opf1 = _block(params["pmt_to_flash"], pmt1, opf, e_pmt_opf, N_OPF)
    i1 = _block(params["sp_to_evt"], n1, i, e_sp_evt, N_EVT) \
        + _block(params["opf_to_evt"], opf1, i, e_opf_evt, N_EVT)
    n2 = _block(params["evt_to_sp"], i1, n1, e_evt_sp, N_SP)
    p1 = _block(params["sp_to_hit"], n2, p, e_sp_hit, N_HIT)
    opf2 = _block(params["evt_to_opf"], i1, opf1, e_evt_opf, N_OPF)
    pmt2 = _block(params["opf_to_pmt"], opf2, pmt1, e_opf_pmt, N_PMT)
    oph1 = _block(params["pmt_to_oph"], pmt2, oph, e_pmt_oph, N_OPH)
    return (p1, n2, oph1, pmt2, opf2, i1)
